# Initial kernel scaffold; baseline (speedup 1.0000x reference)
import jax
import jax.numpy as jnp
from jax import lax
from jax.experimental import pallas as pl
from jax.experimental.pallas import tpu as pltpu
from jax.experimental.pallas import tpu_sc as plsc

NC, NS = 2, 16
NW = NC * NS
B, L, D = 16384, 50, 64
BW = B // NW
CB = 8
NCHUNK = BW // CB
RPC = CB * L
GPIECES = [(0, 128), (128, 128), (256, 128), (384, 16)]


def _body(c_hbm, ctx_hbm, v_hbm, u_hbm, out_hbm,
          cidx, ctxidx, vrows, urows, obuf, sem):
    wid = lax.axis_index("s") * NC + lax.axis_index("c")
    base = wid * BW

    pltpu.sync_copy(c_hbm.at[pl.ds(base, BW)], cidx)
    pltpu.sync_copy(ctx_hbm.at[pl.ds(base * L, BW * L)], ctxidx)

    for j in range(BW // 128):
        pltpu.async_copy(
            v_hbm.at[cidx.at[pl.ds(j * 128, 128)]],
            vrows.at[pl.ds(j * 128, 128)], sem).wait()

    def chunk_body(ch, carry):
        cps = []
        for off, ln in GPIECES:
            src_off = pl.multiple_of(ch * RPC + off, 8)
            cps.append(pltpu.async_copy(
                u_hbm.at[ctxidx.at[pl.ds(src_off, ln)]],
                urows.at[pl.ds(off, ln)], sem))
        for cp in cps:
            cp.wait()

        for ci in range(CB):
            vvec = [vrows[ch * CB + ci, pl.ds(k * 16, 16)] for k in range(4)]

            def l_body(l, carry2, vvec=vvec, ci=ci):
                row = ci * L + l
                acc = vvec[0] * urows[row, pl.ds(0, 16)]
                for k in range(1, 4):
                    acc = acc + vvec[k] * urows[row, pl.ds(k * 16, 16)]
                obuf[row] = jnp.sum(acc)
                return carry2

            lax.fori_loop(0, L, l_body, None)

        dst_off = pl.multiple_of(base * L + ch * RPC, 8)
        pltpu.sync_copy(obuf, out_hbm.at[pl.ds(dst_off, RPC)])
        return carry

    lax.fori_loop(0, NCHUNK, chunk_body, None)


def kernel(center, context_negative, V, U):
    c_flat = center.reshape(B).astype(jnp.int32)
    ctx_flat = context_negative.reshape(B * L).astype(jnp.int32)
    mesh = plsc.VectorSubcoreMesh(core_axis_name="c", subcore_axis_name="s")
    out = pl.kernel(
        _body,
        out_type=jax.ShapeDtypeStruct((B * L,), jnp.float32),
        mesh=mesh,
        scratch_types=[
            pltpu.VMEM((BW,), jnp.int32),
            pltpu.VMEM((BW * L,), jnp.int32),
            pltpu.VMEM((BW, D), jnp.float32),
            pltpu.VMEM((RPC, D), jnp.float32),
            pltpu.VMEM((RPC,), jnp.float32),
            pltpu.SemaphoreType.DMA,
        ],
    )(c_flat, ctx_flat, V, U)
    return out.reshape(B, 1, L)


# speedup vs baseline: 1.5498x; 1.5498x over previous
import jax
import jax.numpy as jnp
from jax import lax
from jax.experimental import pallas as pl
from jax.experimental.pallas import tpu as pltpu
from jax.experimental.pallas import tpu_sc as plsc

NC, NS = 2, 16
NW = NC * NS
B, L, D = 16384, 50, 64
BW = B // NW
CB = 8
NCHUNK = BW // CB
RPC = CB * L
RPAD = RPC + 16
OPAD = 56
GPIECES = [(0, 128), (128, 128), (256, 128), (384, 16)]
VPIECES = [(0, 128), (128, 128), (256, 128), (384, 128)]
GOFFS = (0, 16, 32, 40)
BITREV = (0, 8, 4, 12, 2, 10, 6, 14, 1, 9, 5, 13, 3, 11, 7, 15)


def _worker_id():
    return lax.axis_index("s") * NC + lax.axis_index("c")


def _hsum16(accs, masks, perms):
    cur = [accs[p] for p in BITREV]
    for (m, perm) in zip(masks, perms):
        nxt = []
        for i in range(len(cur) // 2):
            a, b = cur[2 * i], cur[2 * i + 1]
            x = jnp.where(m, b, a)
            y = jnp.where(m, a, b).at[perm].get(
                mode="promise_in_bounds", unique_indices=True)
            nxt.append(x + y)
        cur = nxt
    return cur[0]


def _body(c_hbm, ctx_hbm, v_hbm, u_hbm, out_hbm, *scratch):
    nv, ng = len(VPIECES), len(GPIECES)
    vidx = scratch[:nv]
    gidx = scratch[nv:nv + ng]
    vrows, urows, obuf, sem = scratch[nv + ng:]

    wid = _worker_id()
    base = wid * BW

    lanes = lax.iota(jnp.int32, 16)
    masks = [(lanes & k) != 0 for k in (8, 4, 2, 1)]
    perms = [lanes ^ k for k in (8, 4, 2, 1)]

    for (off, ln), ib in zip(VPIECES, vidx):
        pltpu.sync_copy(c_hbm.at[pl.ds(base + off, ln)], ib)
    vcps = [
        pltpu.async_copy(v_hbm.at[ib], vrows.at[pl.ds(off, ln)], sem)
        for (off, ln), ib in zip(VPIECES, vidx)
    ]
    for cp in vcps:
        cp.wait()

    def chunk_body(ch, carry):
        for (off, ln), ib in zip(GPIECES, gidx):
            src = pl.multiple_of(base * L + ch * RPC + off, 8)
            pltpu.sync_copy(ctx_hbm.at[pl.ds(src, ln)], ib)
        cps = [
            pltpu.async_copy(u_hbm.at[ib], urows.at[pl.ds(off, ln)], sem)
            for (off, ln), ib in zip(GPIECES, gidx)
        ]
        for cp in cps:
            cp.wait()

        def ci_body(ci, carry2):
            vvec = [vrows[ch * CB + ci, pl.ds(k * 16, 16)] for k in range(4)]
            rbase = ci * L
            for goff in GOFFS:
                accs = []
                for j in range(16):
                    row = rbase + goff + j
                    acc = vvec[0] * urows[row, pl.ds(0, 16)]
                    for k in range(1, 4):
                        acc = acc + vvec[k] * urows[row, pl.ds(k * 16, 16)]
                    accs.append(acc)
                obuf[ci, pl.ds(goff, 16)] = _hsum16(accs, masks, perms)
            return carry2

        lax.fori_loop(0, CB, ci_body, None)

        dst = pl.multiple_of(base + ch * CB, 8)
        pltpu.sync_copy(obuf, out_hbm.at[pl.ds(dst, CB), :])
        return carry

    lax.fori_loop(0, NCHUNK, chunk_body, None)


def kernel(center, context_negative, V, U):
    c_flat = center.reshape(B).astype(jnp.int32)
    ctx_flat = context_negative.reshape(B * L).astype(jnp.int32)
    mesh = plsc.VectorSubcoreMesh(
        core_axis_name="c", subcore_axis_name="s",
        num_cores=NC, num_subcores=NS)
    scratch = (
        [pltpu.VMEM((ln,), jnp.int32) for _, ln in VPIECES]
        + [pltpu.VMEM((ln,), jnp.int32) for _, ln in GPIECES]
        + [
            pltpu.VMEM((BW, D), jnp.float32),
            pltpu.VMEM((RPAD, D), jnp.float32),
            pltpu.VMEM((CB, OPAD), jnp.float32),
            pltpu.SemaphoreType.DMA,
        ]
    )
    out = pl.kernel(
        _body,
        out_type=jax.ShapeDtypeStruct((B, OPAD), jnp.float32),
        mesh=mesh,
        scratch_types=scratch,
        compiler_params=pltpu.CompilerParams(use_tc_tiling_on_sc=False),
    )(c_flat, ctx_flat, V, U)
    return out[:, :L].reshape(B, 1, L)
